# baseline (device time: 12475 ns/iter reference)
import jax
import jax.numpy as jnp
from jax import lax
from jax.experimental import pallas as pl
from jax.experimental.pallas import tpu as pltpu

N_DEV = 16
EPS = 1e-5
N_CHUNKS = 2


def kernel(x, gamma):
    m, n_per = x.shape
    n_global = n_per * N_DEV
    gamma2d = gamma.reshape(1, n_per)

    m_sub = m // 128
    rows = m // N_CHUNKS
    sub = m_sub // N_CHUNKS

    def body(x_hbm, g_ref, out_hbm, xv_ref, outv_ref, acc_ref,
             send_sems, recv_sems, load_sems, store_sems):
        my = lax.axis_index("i")

        barrier_sem = pltpu.get_barrier_semaphore()
        for k in range(1, N_DEV):
            pl.semaphore_signal(
                barrier_sem, inc=1,
                device_id=(lax.rem(my + k, N_DEV),),
                device_id_type=pl.DeviceIdType.MESH,
            )

        loads = []
        for c in range(N_CHUNKS):
            cp = pltpu.make_async_copy(
                x_hbm.at[pl.ds(c * rows, rows), :],
                xv_ref.at[pl.ds(c * rows, rows), :],
                load_sems.at[c],
            )
            cp.start()
            loads.append(cp)
        for c in range(N_CHUNKS):
            loads[c].wait()
            xc = xv_ref[c * rows:(c + 1) * rows, :].reshape(sub, 128, n_per)
            acc_ref[0, c * sub:(c + 1) * sub, :] = jnp.sum(xc * xc, axis=2)

        pl.semaphore_wait(barrier_sem, N_DEV - 1)

        rdmas = []
        for k in range(1, N_DEV):
            tgt = lax.rem(my + k, N_DEV)
            rdma = pltpu.make_async_remote_copy(
                src_ref=acc_ref.at[0],
                dst_ref=acc_ref.at[k],
                send_sem=send_sems.at[k],
                recv_sem=recv_sems.at[k],
                device_id=(tgt,),
                device_id_type=pl.DeviceIdType.MESH,
            )
            rdma.start()
            rdmas.append(rdma)

        for rdma in rdmas:
            rdma.wait_recv()

        total = jnp.sum(acc_ref[...], axis=0)
        inv = lax.rsqrt(total / n_global + EPS)

        stores = []
        for c in range(N_CHUNKS):
            xc = xv_ref[c * rows:(c + 1) * rows, :].reshape(sub, 128, n_per)
            invc = inv[c * sub:(c + 1) * sub, :]
            oc = xc * g_ref[...].reshape(1, 1, n_per) * invc[:, :, None]
            outv_ref[c * rows:(c + 1) * rows, :] = oc.reshape(rows, n_per)
            cp = pltpu.make_async_copy(
                outv_ref.at[pl.ds(c * rows, rows), :],
                out_hbm.at[pl.ds(c * rows, rows), :],
                store_sems.at[c],
            )
            cp.start()
            stores.append(cp)
        for cp in stores:
            cp.wait()

        for rdma in rdmas:
            rdma.wait_send()

    return pl.pallas_call(
        body,
        out_shape=jax.ShapeDtypeStruct((m, n_per), x.dtype),
        in_specs=[
            pl.BlockSpec(memory_space=pl.ANY),
            pl.BlockSpec(memory_space=pltpu.VMEM),
        ],
        out_specs=pl.BlockSpec(memory_space=pl.ANY),
        scratch_shapes=[
            pltpu.VMEM((m, n_per), jnp.float32),
            pltpu.VMEM((m, n_per), jnp.float32),
            pltpu.VMEM((N_DEV, m // 128, 128), jnp.float32),
            pltpu.SemaphoreType.DMA((N_DEV,)),
            pltpu.SemaphoreType.DMA((N_DEV,)),
            pltpu.SemaphoreType.DMA((N_CHUNKS,)),
            pltpu.SemaphoreType.DMA((N_CHUNKS,)),
        ],
        compiler_params=pltpu.CompilerParams(collective_id=0),
    )(x, gamma2d)


# device time: 12283 ns/iter; 1.0156x vs baseline; 1.0156x over previous
import jax
import jax.numpy as jnp
from jax import lax
from jax.experimental import pallas as pl
from jax.experimental.pallas import tpu as pltpu

N_DEV = 16
EPS = 1e-5


def kernel(x, gamma):
    m, n_per = x.shape
    n_global = n_per * N_DEV
    gamma2d = gamma.reshape(1, n_per)

    m_sub = m // 128

    def body(x_ref, g_ref, out_ref, acc_ref, send_sems, recv_sems):
        my = lax.axis_index("i")

        barrier_sem = pltpu.get_barrier_semaphore()
        for k in range(1, N_DEV):
            pl.semaphore_signal(
                barrier_sem, inc=1,
                device_id=(lax.rem(my + k, N_DEV),),
                device_id_type=pl.DeviceIdType.MESH,
            )

        x3 = x_ref[...].reshape(m_sub, 128, n_per)
        part = jnp.sum(x3 * x3, axis=2)
        acc_ref[0, :, :] = part

        pl.semaphore_wait(barrier_sem, N_DEV - 1)

        rdmas = []
        for k in range(1, N_DEV):
            tgt = lax.rem(my + k, N_DEV)
            rdma = pltpu.make_async_remote_copy(
                src_ref=acc_ref.at[0],
                dst_ref=acc_ref.at[k],
                send_sem=send_sems.at[k],
                recv_sem=recv_sems.at[k],
                device_id=(tgt,),
                device_id_type=pl.DeviceIdType.MESH,
            )
            rdma.start()
            rdmas.append(rdma)

        for rdma in rdmas:
            rdma.wait_recv()

        total = jnp.sum(acc_ref[...], axis=0)
        inv = lax.rsqrt(total / n_global + EPS)
        out3 = x3 * g_ref[...].reshape(1, 1, n_per) * inv[:, :, None]
        out_ref[...] = out3.reshape(m, n_per)

        for rdma in rdmas:
            rdma.wait_send()

    return pl.pallas_call(
        body,
        out_shape=jax.ShapeDtypeStruct((m, n_per), x.dtype),
        in_specs=[
            pl.BlockSpec(memory_space=pltpu.VMEM),
            pl.BlockSpec(memory_space=pltpu.VMEM),
        ],
        out_specs=pl.BlockSpec(memory_space=pltpu.VMEM),
        scratch_shapes=[
            pltpu.VMEM((N_DEV, m // 128, 128), jnp.float32),
            pltpu.SemaphoreType.DMA((N_DEV,)),
            pltpu.SemaphoreType.DMA((N_DEV,)),
        ],
        compiler_params=pltpu.CompilerParams(collective_id=0),
    )(x, gamma2d)
